# baseline (speedup 1.0000x reference)
import jax
import jax.numpy as jnp
from jax import lax
from jax.experimental import pallas as pl


def _plane_body(ct_ref, rt_ref, out_ref):
    ct = ct_ref[...]
    rt = rt_ref[...]
    p = lax.broadcasted_iota(jnp.int32, (32, 1024), 1)
    k = lax.broadcasted_iota(jnp.int32, (32, 1024), 0)
    tile32 = (p % 32 == k).astype(jnp.float32)
    rep32 = (p // 32 == k).astype(jnp.float32)
    out_ref[:256, :] = jnp.dot(ct, tile32, preferred_element_type=jnp.float32)
    out_ref[256:, :] = jnp.dot(rt, rep32, preferred_element_type=jnp.float32)


def kernel(x, row_embed, col_embed):
    b = x.shape[0]
    h, w = x.shape[-2], x.shape[-1]
    d = row_embed.shape[1]
    ceT = col_embed[:w].T
    reT = row_embed[:h].T
    out = pl.pallas_call(
        _plane_body,
        grid=(b,),
        in_specs=[
            pl.BlockSpec((d, w), lambda i: (0, 0)),
            pl.BlockSpec((d, h), lambda i: (0, 0)),
        ],
        out_specs=pl.BlockSpec((2 * d, h * w), lambda i: (i, 0)),
        out_shape=jax.ShapeDtypeStruct((b * 2 * d, h * w), jnp.float32),
    )(ceT, reT)
    return out.reshape(b, 2 * d, h, w)
